# baseline (speedup 1.0000x reference)
import functools

import jax
import jax.numpy as jnp
from jax import lax
from jax.experimental import pallas as pl
from jax.experimental.pallas import tpu as pltpu
from jax.experimental.pallas import tpu_sc as plsc

N = 4096
G = 128
K = 16
D = 28
DP = 32
RBF = 8
RC = 5.0
ZMAX = 100
RB = 128
NBLK = N // RB
EB = 2048
BIG = 1e10
NW = 32
CH = (N * K) // NW // 128


def _topk_kernel(scal_ref, posr_ref, post_ref, batr_ref, batt_ref, z_ref,
                 emb_ref, bd2_ref, bid_ref, nodes_ref):
    i = pl.program_id(0)
    ws_blk = scal_ref[0, i]
    nch = scal_ref[1, i]

    rowid = i * RB + lax.broadcasted_iota(jnp.int32, (RB, 1), 0)
    br = batr_ref[...]
    px = posr_ref[...]

    def cand_d2(c):
        off = (ws_blk + c) * RB
        pc = post_ref[:, pl.ds(off, RB)]
        d2 = ((px[:, 0:1] - pc[0:1, :]) ** 2
              + (px[:, 1:2] - pc[1:2, :]) ** 2
              + (px[:, 2:3] - pc[2:3, :]) ** 2)
        bc = batt_ref[:, pl.ds(off, RB)]
        gcol = off + lax.broadcasted_iota(jnp.int32, (1, RB), 1)
        mask = (br == bc) & (rowid != gcol)
        return jnp.where(mask, d2, BIG), off

    bd2_ref[...] = jnp.full((RB, K), BIG, jnp.float32)

    def chunk_body(c, carry):
        cand, _ = cand_d2(c)
        comb = jnp.concatenate([bd2_ref[...], cand], axis=1)
        ms = []
        for k in range(K):
            m = jnp.min(comb, axis=1, keepdims=True)
            ms.append(m)
            comb = jnp.where(comb == m, BIG, comb)
        bd2_ref[...] = jnp.concatenate(ms, axis=1)
        return carry

    lax.fori_loop(0, nch, chunk_body, 0)

    nid_ref = bid_ref
    nid_ref[...] = jnp.zeros((RB, K), jnp.int32)
    bd = bd2_ref[...]

    def id_body(c, carry):
        cand, off = cand_d2(c)
        gcolc = (off + 1 + lax.broadcasted_iota(jnp.int32, (RB, 1), 0)
                 ).astype(jnp.float32)
        ids = []
        for k in range(K):
            mk = (cand == bd[:, k:k + 1]).astype(jnp.float32)
            ids.append(jnp.dot(mk, gcolc,
                               precision=lax.Precision.HIGHEST,
                               preferred_element_type=jnp.float32))
        idcat = jnp.concatenate(ids, axis=1).astype(jnp.int32)
        cur = nid_ref[...]
        nid_ref[...] = jnp.where(cur == 0, idcat, cur)
        return carry

    lax.fori_loop(0, nch, id_body, 0)
    bid_ref[...] = jnp.clip(nid_ref[...] - 1, 0, N - 1)

    zoh = (z_ref[...] == lax.broadcasted_iota(jnp.int32, (1, 128), 1)
           ).astype(jnp.float32)
    nodes_ref[...] = jnp.dot(zoh, emb_ref[...],
                             precision=lax.Precision.HIGHEST,
                             preferred_element_type=jnp.float32)


def _geom_kernel(d2_ref, dist_ref, g_ref):
    d2 = d2_ref[...]
    dist = jnp.sqrt(d2 + 1e-12)
    dc = jnp.minimum(dist, RC)
    fc = 0.5 * (jnp.cos(jnp.pi * dc / RC) + 1.0)
    valid = (d2 < 1e9).astype(jnp.float32)
    dist_ref[...] = dist
    g_ref[...] = fc * valid


def _edge_kernel(dist_ref, g_ref, w1_ref, b1_ref, w2_ref, out_ref):
    dist = dist_ref[...]
    cen = (RC / (RBF - 1)) * lax.broadcasted_iota(jnp.int32, (1, RBF), 1
                                                  ).astype(jnp.float32)
    rbf = jnp.exp(-10.0 * (dist - cen) ** 2)
    h = jax.nn.silu(jnp.dot(rbf, w1_ref[...],
                            preferred_element_type=jnp.float32) + b1_ref[...])
    we = jnp.dot(h, w2_ref[...], preferred_element_type=jnp.float32)
    out_ref[...] = we * g_ref[...]


def _interact_kernel(nbr_ref, wfc_ref, h_ref, ws_ref, wm_ref, out_ref):
    prod = nbr_ref[...] * wfc_ref[...]
    msg = jnp.sum(prod.reshape(RB, K, DP), axis=1)
    h = h_ref[...]
    out_ref[...] = jax.nn.silu(
        jnp.dot(h, ws_ref[...], preferred_element_type=jnp.float32)
        + jnp.dot(msg, wm_ref[...], preferred_element_type=jnp.float32))


def _final_kernel(nbr_ref, wfc_ref, h_ref, ws_ref, wm_ref, w1_ref, w2_ref,
                  z_ref, shift_ref, batr_ref, out_ref):
    i = pl.program_id(0)
    prod = nbr_ref[...] * wfc_ref[...]
    msg = jnp.sum(prod.reshape(RB, K, DP), axis=1)
    h = h_ref[...]
    h2 = jax.nn.silu(
        jnp.dot(h, ws_ref[...], preferred_element_type=jnp.float32)
        + jnp.dot(msg, wm_ref[...], preferred_element_type=jnp.float32))
    t = jax.nn.silu(jnp.dot(h2, w1_ref[...],
                            preferred_element_type=jnp.float32))
    ae = jnp.dot(t, w2_ref[...], preferred_element_type=jnp.float32)
    zoh = (z_ref[...] == lax.broadcasted_iota(jnp.int32, (1, 128), 1)
           ).astype(jnp.float32)
    ae = ae + jnp.dot(zoh, shift_ref[...], precision=lax.Precision.HIGHEST,
                      preferred_element_type=jnp.float32)
    boh = (batr_ref[...] == lax.broadcasted_iota(jnp.int32, (1, G), 1)
           ).astype(jnp.float32)
    contrib = jnp.sum(boh * ae, axis=0, keepdims=True)

    @pl.when(i == 0)
    def _():
        out_ref[...] = jnp.zeros((1, G), jnp.float32)

    out_ref[...] += contrib


def _sc_gather(table, idx3):
    mesh = plsc.VectorSubcoreMesh(core_axis_name="c", subcore_axis_name="s")

    @functools.partial(
        pl.kernel,
        out_type=jax.ShapeDtypeStruct((NW * CH * 128, DP), jnp.float32),
        mesh=mesh,
        compiler_params=pltpu.CompilerParams(use_tc_tiling_on_sc=False),
        scratch_types=[
            pltpu.VMEM((CH, 128), jnp.int32),
            pltpu.VMEM((CH * 128, DP), jnp.float32),
            pltpu.SemaphoreType.DMA,
        ],
    )
    def gk(table_hbm, idx_hbm, out_hbm, idx_v, rows_v, sem):
        w = lax.axis_index("s") * 2 + lax.axis_index("c")
        pltpu.sync_copy(idx_hbm.at[w], idx_v)
        copies = [
            pltpu.async_copy(table_hbm.at[idx_v.at[j]],
                             rows_v.at[pl.ds(j * 128, 128)], sem)
            for j in range(CH)
        ]
        for c in copies:
            c.wait()
        pltpu.sync_copy(rows_v, out_hbm.at[pl.ds(w * CH * 128, CH * 128)])

    return gk(table, idx3)


def kernel(z, pos, batch, emb, W_rbf1, b_rbf1, W_rbf2, W_self, W_msg, W1, W2,
           shift):
    pos = pos.astype(jnp.float32)
    posT = pos.T
    batch = batch.astype(jnp.int32)
    batr = batch.reshape(N, 1)
    batt = batch.reshape(1, N)
    z2 = z.astype(jnp.int32).reshape(N, 1)

    emb_p = jnp.zeros((128, DP), jnp.float32).at[:ZMAX, :D].set(emb)
    w2rbf_p = jnp.zeros((32, DP), jnp.float32).at[:, :D].set(W_rbf2)
    wself_p = jnp.zeros((DP, DP), jnp.float32).at[:D, :D].set(W_self)
    wmsg_p = jnp.zeros((DP, DP), jnp.float32).at[:D, :D].set(W_msg)
    w1_p = jnp.zeros((DP, 32), jnp.float32).at[:D, :].set(W1)
    shift_p = jnp.zeros((128, 1), jnp.float32).at[:ZMAX, :].set(shift)
    b1 = b_rbf1.reshape(1, 32)

    gids = jnp.arange(G, dtype=jnp.int32)
    starts = jnp.searchsorted(batch, gids, side="left").astype(jnp.int32)
    ends = jnp.searchsorted(batch, gids, side="right").astype(jnp.int32)
    bfirst = batch.reshape(NBLK, RB)[:, 0]
    blast = batch.reshape(NBLK, RB)[:, -1]
    ws_blk = starts[bfirst] // RB
    nch = (ends[blast] - 1) // RB - ws_blk + 1
    scal = jnp.stack([ws_blk, nch]).astype(jnp.int32)

    full = lambda shape: pl.BlockSpec(shape, lambda i, s: (0, 0))
    rowblk = lambda shape: pl.BlockSpec(shape, lambda i, s: (i, 0))

    bd2, bid, nodes = pl.pallas_call(
        _topk_kernel,
        grid_spec=pltpu.PrefetchScalarGridSpec(
            num_scalar_prefetch=1,
            grid=(NBLK,),
            in_specs=[
                rowblk((RB, 3)),
                full((3, N)),
                rowblk((RB, 1)),
                full((1, N)),
                rowblk((RB, 1)),
                full((128, DP)),
            ],
            out_specs=[
                rowblk((RB, K)),
                rowblk((RB, K)),
                rowblk((RB, DP)),
            ],
        ),
        out_shape=[
            jax.ShapeDtypeStruct((N, K), jnp.float32),
            jax.ShapeDtypeStruct((N, K), jnp.int32),
            jax.ShapeDtypeStruct((N, DP), jnp.float32),
        ],
    )(scal, pos, posT, batr, batt, z2, emb_p)

    d2w = bd2.reshape(N * K // 128, 128)
    distw, gw = pl.pallas_call(
        _geom_kernel,
        in_specs=[pl.BlockSpec(d2w.shape, lambda: (0, 0))],
        out_specs=[pl.BlockSpec(d2w.shape, lambda: (0, 0)),
                   pl.BlockSpec(d2w.shape, lambda: (0, 0))],
        out_shape=[jax.ShapeDtypeStruct(d2w.shape, jnp.float32),
                   jax.ShapeDtypeStruct(d2w.shape, jnp.float32)],
    )(d2w)

    diste = distw.reshape(N * K, 1)
    ge = gw.reshape(N * K, 1)
    wfc = pl.pallas_call(
        _edge_kernel,
        grid=((N * K) // EB,),
        in_specs=[
            pl.BlockSpec((EB, 1), lambda i: (i, 0)),
            pl.BlockSpec((EB, 1), lambda i: (i, 0)),
            pl.BlockSpec((RBF, 32), lambda i: (0, 0)),
            pl.BlockSpec((1, 32), lambda i: (0, 0)),
            pl.BlockSpec((32, DP), lambda i: (0, 0)),
        ],
        out_specs=pl.BlockSpec((EB, DP), lambda i: (i, 0)),
        out_shape=jax.ShapeDtypeStruct((N * K, DP), jnp.float32),
    )(diste, ge, W_rbf1, b1, w2rbf_p)

    idx3 = bid.reshape(NW, CH, 128)

    def interact(h, final):
        nbr = _sc_gather(h, idx3)
        if not final:
            return pl.pallas_call(
                _interact_kernel,
                grid=(NBLK,),
                in_specs=[
                    pl.BlockSpec((RB * K, DP), lambda i: (i, 0)),
                    pl.BlockSpec((RB * K, DP), lambda i: (i, 0)),
                    pl.BlockSpec((RB, DP), lambda i: (i, 0)),
                    pl.BlockSpec((DP, DP), lambda i: (0, 0)),
                    pl.BlockSpec((DP, DP), lambda i: (0, 0)),
                ],
                out_specs=pl.BlockSpec((RB, DP), lambda i: (i, 0)),
                out_shape=jax.ShapeDtypeStruct((N, DP), jnp.float32),
            )(nbr, wfc, h, wself_p, wmsg_p)
        return pl.pallas_call(
            _final_kernel,
            grid=(NBLK,),
            in_specs=[
                pl.BlockSpec((RB * K, DP), lambda i: (i, 0)),
                pl.BlockSpec((RB * K, DP), lambda i: (i, 0)),
                pl.BlockSpec((RB, DP), lambda i: (i, 0)),
                pl.BlockSpec((DP, DP), lambda i: (0, 0)),
                pl.BlockSpec((DP, DP), lambda i: (0, 0)),
                pl.BlockSpec((DP, 32), lambda i: (0, 0)),
                pl.BlockSpec((32, 1), lambda i: (0, 0)),
                pl.BlockSpec((RB, 1), lambda i: (i, 0)),
                pl.BlockSpec((128, 1), lambda i: (0, 0)),
                pl.BlockSpec((RB, 1), lambda i: (i, 0)),
            ],
            out_specs=pl.BlockSpec((1, G), lambda i: (0, 0)),
            out_shape=jax.ShapeDtypeStruct((1, G), jnp.float32),
        )(nbr, wfc, h, wself_p, wmsg_p, w1_p, W2, z2, shift_p, batr)

    h1 = interact(nodes, final=False)
    eng = interact(h1, final=True)
    return eng.reshape(G, 1)


# speedup vs baseline: 1.0192x; 1.0192x over previous
import functools

import jax
import jax.numpy as jnp
from jax import lax
from jax.experimental import pallas as pl
from jax.experimental.pallas import tpu as pltpu
from jax.experimental.pallas import tpu_sc as plsc

N = 4096
G = 128
K = 16
D = 28
DP = 32
RBF = 8
RC = 5.0
ZMAX = 100
RB = 128
NBLK = N // RB
RBR = 256
CW = 256
NBLKR = N // RBR
EB = 2048
BIG = 1e10
NW = 32
CH = (N * K) // NW // 128


def _topk_kernel(scal_ref, posr_ref, post_ref, batr_ref, batt_ref, z_ref,
                 emb_ref, bd2_ref, bid_ref, nodes_ref):
    i = pl.program_id(0)
    ws_blk = scal_ref[0, i]
    nch = scal_ref[1, i]

    rowid = i * RBR + lax.broadcasted_iota(jnp.int32, (RBR, 1), 0)
    br = batr_ref[...]
    px = posr_ref[...]

    def cand_d2(c):
        off = (ws_blk + c) * CW
        pc = post_ref[:, pl.ds(off, CW)]
        d2 = ((px[:, 0:1] - pc[0:1, :]) ** 2
              + (px[:, 1:2] - pc[1:2, :]) ** 2
              + (px[:, 2:3] - pc[2:3, :]) ** 2)
        bc = batt_ref[:, pl.ds(off, CW)]
        gcol = off + lax.broadcasted_iota(jnp.int32, (1, CW), 1)
        mask = (br == bc) & (rowid != gcol)
        return jnp.where(mask, d2, BIG), off

    bd2_ref[...] = jnp.full((RBR, K), BIG, jnp.float32)

    def chunk_body(c, carry):
        cand, _ = cand_d2(c)
        comb = jnp.concatenate([bd2_ref[...], cand], axis=1)
        ms = []
        for k in range(K):
            m = jnp.min(comb, axis=1, keepdims=True)
            ms.append(m)
            comb = jnp.where(comb == m, BIG, comb)
        bd2_ref[...] = jnp.concatenate(ms, axis=1)
        return carry

    lax.fori_loop(0, nch, chunk_body, 0)

    nid_ref = bid_ref
    nid_ref[...] = jnp.zeros((RBR, K), jnp.int32)
    bd = bd2_ref[...]

    def id_body(c, carry):
        cand, off = cand_d2(c)
        gcolc = (off + 1 + lax.broadcasted_iota(jnp.int32, (CW, 1), 0)
                 ).astype(jnp.float32)
        ids = []
        for k in range(K):
            mk = (cand == bd[:, k:k + 1]).astype(jnp.float32)
            ids.append(jnp.dot(mk, gcolc,
                               precision=lax.Precision.HIGHEST,
                               preferred_element_type=jnp.float32))
        idcat = jnp.concatenate(ids, axis=1).astype(jnp.int32)
        cur = nid_ref[...]
        nid_ref[...] = jnp.where(cur == 0, idcat, cur)
        return carry

    lax.fori_loop(0, nch, id_body, 0)
    bid_ref[...] = jnp.clip(nid_ref[...] - 1, 0, N - 1)

    zoh = (z_ref[...] == lax.broadcasted_iota(jnp.int32, (1, 128), 1)
           ).astype(jnp.float32)
    nodes_ref[...] = jnp.dot(zoh, emb_ref[...],
                             precision=lax.Precision.HIGHEST,
                             preferred_element_type=jnp.float32)


def _geom_kernel(d2_ref, dist_ref, g_ref):
    d2 = d2_ref[...]
    dist = jnp.sqrt(d2 + 1e-12)
    dc = jnp.minimum(dist, RC)
    fc = 0.5 * (jnp.cos(jnp.pi * dc / RC) + 1.0)
    valid = (d2 < 1e9).astype(jnp.float32)
    dist_ref[...] = dist
    g_ref[...] = fc * valid


def _edge_kernel(dist_ref, g_ref, w1_ref, b1_ref, w2_ref, out_ref):
    dist = dist_ref[...]
    cen = (RC / (RBF - 1)) * lax.broadcasted_iota(jnp.int32, (1, RBF), 1
                                                  ).astype(jnp.float32)
    rbf = jnp.exp(-10.0 * (dist - cen) ** 2)
    h = jax.nn.silu(jnp.dot(rbf, w1_ref[...],
                            preferred_element_type=jnp.float32) + b1_ref[...])
    we = jnp.dot(h, w2_ref[...], preferred_element_type=jnp.float32)
    out_ref[...] = we * g_ref[...]


def _interact_kernel(nbr_ref, wfc_ref, h_ref, ws_ref, wm_ref, out_ref):
    prod = nbr_ref[...] * wfc_ref[...]
    msg = jnp.sum(prod.reshape(RB, K, DP), axis=1)
    h = h_ref[...]
    out_ref[...] = jax.nn.silu(
        jnp.dot(h, ws_ref[...], preferred_element_type=jnp.float32)
        + jnp.dot(msg, wm_ref[...], preferred_element_type=jnp.float32))


def _final_kernel(nbr_ref, wfc_ref, h_ref, ws_ref, wm_ref, w1_ref, w2_ref,
                  z_ref, shift_ref, batr_ref, out_ref):
    i = pl.program_id(0)
    prod = nbr_ref[...] * wfc_ref[...]
    msg = jnp.sum(prod.reshape(RB, K, DP), axis=1)
    h = h_ref[...]
    h2 = jax.nn.silu(
        jnp.dot(h, ws_ref[...], preferred_element_type=jnp.float32)
        + jnp.dot(msg, wm_ref[...], preferred_element_type=jnp.float32))
    t = jax.nn.silu(jnp.dot(h2, w1_ref[...],
                            preferred_element_type=jnp.float32))
    ae = jnp.dot(t, w2_ref[...], preferred_element_type=jnp.float32)
    zoh = (z_ref[...] == lax.broadcasted_iota(jnp.int32, (1, 128), 1)
           ).astype(jnp.float32)
    ae = ae + jnp.dot(zoh, shift_ref[...], precision=lax.Precision.HIGHEST,
                      preferred_element_type=jnp.float32)
    boh = (batr_ref[...] == lax.broadcasted_iota(jnp.int32, (1, G), 1)
           ).astype(jnp.float32)
    contrib = jnp.sum(boh * ae, axis=0, keepdims=True)

    @pl.when(i == 0)
    def _():
        out_ref[...] = jnp.zeros((1, G), jnp.float32)

    out_ref[...] += contrib


def _sc_gather(table, idx3):
    mesh = plsc.VectorSubcoreMesh(core_axis_name="c", subcore_axis_name="s")

    @functools.partial(
        pl.kernel,
        out_type=jax.ShapeDtypeStruct((NW * CH * 128, DP), jnp.float32),
        mesh=mesh,
        compiler_params=pltpu.CompilerParams(use_tc_tiling_on_sc=False),
        scratch_types=[
            pltpu.VMEM((CH, 128), jnp.int32),
            pltpu.VMEM((CH * 128, DP), jnp.float32),
            pltpu.SemaphoreType.DMA,
        ],
    )
    def gk(table_hbm, idx_hbm, out_hbm, idx_v, rows_v, sem):
        w = lax.axis_index("s") * 2 + lax.axis_index("c")
        pltpu.sync_copy(idx_hbm.at[w], idx_v)
        copies = [
            pltpu.async_copy(table_hbm.at[idx_v.at[j]],
                             rows_v.at[pl.ds(j * 128, 128)], sem)
            for j in range(CH)
        ]
        for c in copies:
            c.wait()
        pltpu.sync_copy(rows_v, out_hbm.at[pl.ds(w * CH * 128, CH * 128)])

    return gk(table, idx3)


def kernel(z, pos, batch, emb, W_rbf1, b_rbf1, W_rbf2, W_self, W_msg, W1, W2,
           shift):
    pos = pos.astype(jnp.float32)
    posT = pos.T
    batch = batch.astype(jnp.int32)
    batr = batch.reshape(N, 1)
    batt = batch.reshape(1, N)
    z2 = z.astype(jnp.int32).reshape(N, 1)

    emb_p = jnp.zeros((128, DP), jnp.float32).at[:ZMAX, :D].set(emb)
    w2rbf_p = jnp.zeros((32, DP), jnp.float32).at[:, :D].set(W_rbf2)
    wself_p = jnp.zeros((DP, DP), jnp.float32).at[:D, :D].set(W_self)
    wmsg_p = jnp.zeros((DP, DP), jnp.float32).at[:D, :D].set(W_msg)
    w1_p = jnp.zeros((DP, 32), jnp.float32).at[:D, :].set(W1)
    shift_p = jnp.zeros((128, 1), jnp.float32).at[:ZMAX, :].set(shift)
    b1 = b_rbf1.reshape(1, 32)

    gids = jnp.arange(G, dtype=jnp.int32)
    starts = jnp.searchsorted(batch, gids, side="left").astype(jnp.int32)
    ends = jnp.searchsorted(batch, gids, side="right").astype(jnp.int32)
    bfirst = batch.reshape(NBLKR, RBR)[:, 0]
    blast = batch.reshape(NBLKR, RBR)[:, -1]
    ws_blk = starts[bfirst] // CW
    nch = (ends[blast] - 1) // CW - ws_blk + 1
    scal = jnp.stack([ws_blk, nch]).astype(jnp.int32)

    full = lambda shape: pl.BlockSpec(shape, lambda i, s: (0, 0))
    rowblk = lambda shape: pl.BlockSpec(shape, lambda i, s: (i, 0))

    bd2, bid, nodes = pl.pallas_call(
        _topk_kernel,
        grid_spec=pltpu.PrefetchScalarGridSpec(
            num_scalar_prefetch=1,
            grid=(NBLKR,),
            in_specs=[
                rowblk((RBR, 3)),
                full((3, N)),
                rowblk((RBR, 1)),
                full((1, N)),
                rowblk((RBR, 1)),
                full((128, DP)),
            ],
            out_specs=[
                rowblk((RBR, K)),
                rowblk((RBR, K)),
                rowblk((RBR, DP)),
            ],
        ),
        out_shape=[
            jax.ShapeDtypeStruct((N, K), jnp.float32),
            jax.ShapeDtypeStruct((N, K), jnp.int32),
            jax.ShapeDtypeStruct((N, DP), jnp.float32),
        ],
    )(scal, pos, posT, batr, batt, z2, emb_p)

    d2w = bd2.reshape(N * K // 128, 128)
    distw, gw = pl.pallas_call(
        _geom_kernel,
        in_specs=[pl.BlockSpec(d2w.shape, lambda: (0, 0))],
        out_specs=[pl.BlockSpec(d2w.shape, lambda: (0, 0)),
                   pl.BlockSpec(d2w.shape, lambda: (0, 0))],
        out_shape=[jax.ShapeDtypeStruct(d2w.shape, jnp.float32),
                   jax.ShapeDtypeStruct(d2w.shape, jnp.float32)],
    )(d2w)

    diste = distw.reshape(N * K, 1)
    ge = gw.reshape(N * K, 1)
    wfc = pl.pallas_call(
        _edge_kernel,
        grid=((N * K) // EB,),
        in_specs=[
            pl.BlockSpec((EB, 1), lambda i: (i, 0)),
            pl.BlockSpec((EB, 1), lambda i: (i, 0)),
            pl.BlockSpec((RBF, 32), lambda i: (0, 0)),
            pl.BlockSpec((1, 32), lambda i: (0, 0)),
            pl.BlockSpec((32, DP), lambda i: (0, 0)),
        ],
        out_specs=pl.BlockSpec((EB, DP), lambda i: (i, 0)),
        out_shape=jax.ShapeDtypeStruct((N * K, DP), jnp.float32),
    )(diste, ge, W_rbf1, b1, w2rbf_p)

    idx3 = bid.reshape(NW, CH, 128)

    def interact(h, final):
        nbr = _sc_gather(h, idx3)
        if not final:
            return pl.pallas_call(
                _interact_kernel,
                grid=(NBLK,),
                in_specs=[
                    pl.BlockSpec((RB * K, DP), lambda i: (i, 0)),
                    pl.BlockSpec((RB * K, DP), lambda i: (i, 0)),
                    pl.BlockSpec((RB, DP), lambda i: (i, 0)),
                    pl.BlockSpec((DP, DP), lambda i: (0, 0)),
                    pl.BlockSpec((DP, DP), lambda i: (0, 0)),
                ],
                out_specs=pl.BlockSpec((RB, DP), lambda i: (i, 0)),
                out_shape=jax.ShapeDtypeStruct((N, DP), jnp.float32),
            )(nbr, wfc, h, wself_p, wmsg_p)
        return pl.pallas_call(
            _final_kernel,
            grid=(NBLK,),
            in_specs=[
                pl.BlockSpec((RB * K, DP), lambda i: (i, 0)),
                pl.BlockSpec((RB * K, DP), lambda i: (i, 0)),
                pl.BlockSpec((RB, DP), lambda i: (i, 0)),
                pl.BlockSpec((DP, DP), lambda i: (0, 0)),
                pl.BlockSpec((DP, DP), lambda i: (0, 0)),
                pl.BlockSpec((DP, 32), lambda i: (0, 0)),
                pl.BlockSpec((32, 1), lambda i: (0, 0)),
                pl.BlockSpec((RB, 1), lambda i: (i, 0)),
                pl.BlockSpec((128, 1), lambda i: (0, 0)),
                pl.BlockSpec((RB, 1), lambda i: (i, 0)),
            ],
            out_specs=pl.BlockSpec((1, G), lambda i: (0, 0)),
            out_shape=jax.ShapeDtypeStruct((1, G), jnp.float32),
        )(nbr, wfc, h, wself_p, wmsg_p, w1_p, W2, z2, shift_p, batr)

    h1 = interact(nodes, final=False)
    eng = interact(h1, final=True)
    return eng.reshape(G, 1)


# speedup vs baseline: 1.2557x; 1.2320x over previous
import functools

import jax
import jax.numpy as jnp
from jax import lax
from jax.experimental import pallas as pl
from jax.experimental.pallas import tpu as pltpu
from jax.experimental.pallas import tpu_sc as plsc

N = 4096
G = 128
K = 16
D = 28
DP = 32
RBF = 8
RC = 5.0
ZMAX = 100
RB = 128
NBLK = N // RB
RBR = 256
CW = 256
NBLKR = N // RBR
EB = 2048
BIG = 1e10
NW = 32
CH = (N * K) // NW // 128


def _topk_kernel(scal_ref, posr_ref, post_ref, batr_ref, batt_ref, z_ref,
                 emb_ref, bd2_ref, bid_ref, nodes_ref):
    i = pl.program_id(0)
    ws_blk = scal_ref[0, i]
    nch = scal_ref[1, i]

    rowid = i * RBR + lax.broadcasted_iota(jnp.int32, (1, RBR), 1)
    br = batt_ref[:, pl.ds(i * RBR, RBR)]
    prw = post_ref[:, pl.ds(i * RBR, RBR)]

    def cand_d2(c):
        off = (ws_blk + c) * CW
        pc = posr_ref[pl.ds(off, CW), :]
        d2 = ((pc[:, 0:1] - prw[0:1, :]) ** 2
              + (pc[:, 1:2] - prw[1:2, :]) ** 2
              + (pc[:, 2:3] - prw[2:3, :]) ** 2)
        bc = batr_ref[pl.ds(off, CW), :]
        gcol = off + lax.broadcasted_iota(jnp.int32, (CW, 1), 0)
        mask = (bc == br) & (gcol != rowid)
        return jnp.where(mask, d2, BIG), off

    bd2_ref[...] = jnp.full((K, RBR), BIG, jnp.float32)

    def chunk_body(c, carry):
        cand, _ = cand_d2(c)
        comb = jnp.concatenate([bd2_ref[...], cand], axis=0)
        ms = []
        for k in range(K):
            m = jnp.min(comb, axis=0, keepdims=True)
            ms.append(m)
            comb = jnp.where(comb == m, BIG, comb)
        bd2_ref[...] = jnp.concatenate(ms, axis=0)
        return carry

    lax.fori_loop(0, nch, chunk_body, 0)

    nid_ref = bid_ref
    nid_ref[...] = jnp.zeros((K, RBR), jnp.int32)
    bd = bd2_ref[...]

    def id_body(c, carry):
        cand, off = cand_d2(c)
        gcr = (off + 1 + lax.broadcasted_iota(jnp.int32, (1, CW), 1)
               ).astype(jnp.float32)
        ids = []
        for k in range(K):
            mk = (cand == bd[k:k + 1, :]).astype(jnp.float32)
            ids.append(jnp.dot(gcr, mk,
                               precision=lax.Precision.HIGHEST,
                               preferred_element_type=jnp.float32))
        idcat = jnp.concatenate(ids, axis=0).astype(jnp.int32)
        cur = nid_ref[...]
        nid_ref[...] = jnp.where(cur == 0, idcat, cur)
        return carry

    lax.fori_loop(0, nch, id_body, 0)
    bid_ref[...] = jnp.clip(nid_ref[...] - 1, 0, N - 1)

    zoh = (z_ref[...] == lax.broadcasted_iota(jnp.int32, (1, 128), 1)
           ).astype(jnp.float32)
    nodes_ref[...] = jnp.dot(zoh, emb_ref[...],
                             precision=lax.Precision.HIGHEST,
                             preferred_element_type=jnp.float32)


def _geom_kernel(d2_ref, dist_ref, g_ref):
    d2 = d2_ref[...]
    dist = jnp.sqrt(d2 + 1e-12)
    dc = jnp.minimum(dist, RC)
    fc = 0.5 * (jnp.cos(jnp.pi * dc / RC) + 1.0)
    valid = (d2 < 1e9).astype(jnp.float32)
    dist_ref[...] = dist
    g_ref[...] = fc * valid


def _edge_kernel(dist_ref, g_ref, w1_ref, b1_ref, w2_ref, out_ref):
    dist = dist_ref[...]
    cen = (RC / (RBF - 1)) * lax.broadcasted_iota(jnp.int32, (1, RBF), 1
                                                  ).astype(jnp.float32)
    rbf = jnp.exp(-10.0 * (dist - cen) ** 2)
    h = jax.nn.silu(jnp.dot(rbf, w1_ref[...],
                            preferred_element_type=jnp.float32) + b1_ref[...])
    we = jnp.dot(h, w2_ref[...], preferred_element_type=jnp.float32)
    out_ref[...] = we * g_ref[...]


def _interact_kernel(nbr_ref, wfc_ref, h_ref, ws_ref, wm_ref, out_ref):
    prod = nbr_ref[...] * wfc_ref[...]
    msg = jnp.sum(prod.reshape(RB, K, DP), axis=1)
    h = h_ref[...]
    out_ref[...] = jax.nn.silu(
        jnp.dot(h, ws_ref[...], preferred_element_type=jnp.float32)
        + jnp.dot(msg, wm_ref[...], preferred_element_type=jnp.float32))


def _final_kernel(nbr_ref, wfc_ref, h_ref, ws_ref, wm_ref, w1_ref, w2_ref,
                  z_ref, shift_ref, batr_ref, out_ref):
    i = pl.program_id(0)
    prod = nbr_ref[...] * wfc_ref[...]
    msg = jnp.sum(prod.reshape(RB, K, DP), axis=1)
    h = h_ref[...]
    h2 = jax.nn.silu(
        jnp.dot(h, ws_ref[...], preferred_element_type=jnp.float32)
        + jnp.dot(msg, wm_ref[...], preferred_element_type=jnp.float32))
    t = jax.nn.silu(jnp.dot(h2, w1_ref[...],
                            preferred_element_type=jnp.float32))
    ae = jnp.dot(t, w2_ref[...], preferred_element_type=jnp.float32)
    zoh = (z_ref[...] == lax.broadcasted_iota(jnp.int32, (1, 128), 1)
           ).astype(jnp.float32)
    ae = ae + jnp.dot(zoh, shift_ref[...], precision=lax.Precision.HIGHEST,
                      preferred_element_type=jnp.float32)
    boh = (batr_ref[...] == lax.broadcasted_iota(jnp.int32, (1, G), 1)
           ).astype(jnp.float32)
    contrib = jnp.sum(boh * ae, axis=0, keepdims=True)

    @pl.when(i == 0)
    def _():
        out_ref[...] = jnp.zeros((1, G), jnp.float32)

    out_ref[...] += contrib


def _sc_gather(table, idx3):
    mesh = plsc.VectorSubcoreMesh(core_axis_name="c", subcore_axis_name="s")

    @functools.partial(
        pl.kernel,
        out_type=jax.ShapeDtypeStruct((NW * CH * 128, DP), jnp.float32),
        mesh=mesh,
        compiler_params=pltpu.CompilerParams(use_tc_tiling_on_sc=False),
        scratch_types=[
            pltpu.VMEM((CH, 128), jnp.int32),
            pltpu.VMEM((CH * 128, DP), jnp.float32),
            pltpu.SemaphoreType.DMA,
        ],
    )
    def gk(table_hbm, idx_hbm, out_hbm, idx_v, rows_v, sem):
        w = lax.axis_index("s") * 2 + lax.axis_index("c")
        pltpu.sync_copy(idx_hbm.at[w], idx_v)
        copies = [
            pltpu.async_copy(table_hbm.at[idx_v.at[j]],
                             rows_v.at[pl.ds(j * 128, 128)], sem)
            for j in range(CH)
        ]
        for c in copies:
            c.wait()
        pltpu.sync_copy(rows_v, out_hbm.at[pl.ds(w * CH * 128, CH * 128)])

    return gk(table, idx3)


def kernel(z, pos, batch, emb, W_rbf1, b_rbf1, W_rbf2, W_self, W_msg, W1, W2,
           shift):
    pos = pos.astype(jnp.float32)
    posT = pos.T
    batch = batch.astype(jnp.int32)
    batr = batch.reshape(N, 1)
    batt = batch.reshape(1, N)
    z2 = z.astype(jnp.int32).reshape(N, 1)

    emb_p = jnp.zeros((128, DP), jnp.float32).at[:ZMAX, :D].set(emb)
    w2rbf_p = jnp.zeros((32, DP), jnp.float32).at[:, :D].set(W_rbf2)
    wself_p = jnp.zeros((DP, DP), jnp.float32).at[:D, :D].set(W_self)
    wmsg_p = jnp.zeros((DP, DP), jnp.float32).at[:D, :D].set(W_msg)
    w1_p = jnp.zeros((DP, 32), jnp.float32).at[:D, :].set(W1)
    shift_p = jnp.zeros((128, 1), jnp.float32).at[:ZMAX, :].set(shift)
    b1 = b_rbf1.reshape(1, 32)

    gids = jnp.arange(G, dtype=jnp.int32)
    starts = jnp.searchsorted(batch, gids, side="left").astype(jnp.int32)
    ends = jnp.searchsorted(batch, gids, side="right").astype(jnp.int32)
    bfirst = batch.reshape(NBLKR, RBR)[:, 0]
    blast = batch.reshape(NBLKR, RBR)[:, -1]
    ws_blk = starts[bfirst] // CW
    nch = (ends[blast] - 1) // CW - ws_blk + 1
    scal = jnp.stack([ws_blk, nch]).astype(jnp.int32)

    full = lambda shape: pl.BlockSpec(shape, lambda i, s: (0, 0))
    rowblk = lambda shape: pl.BlockSpec(shape, lambda i, s: (i, 0))

    colblk = lambda shape: pl.BlockSpec(shape, lambda i, s: (0, i))
    bd2t, bidt, nodes = pl.pallas_call(
        _topk_kernel,
        grid_spec=pltpu.PrefetchScalarGridSpec(
            num_scalar_prefetch=1,
            grid=(NBLKR,),
            in_specs=[
                full((N, 3)),
                full((3, N)),
                full((N, 1)),
                full((1, N)),
                rowblk((RBR, 1)),
                full((128, DP)),
            ],
            out_specs=[
                colblk((K, RBR)),
                colblk((K, RBR)),
                rowblk((RBR, DP)),
            ],
        ),
        out_shape=[
            jax.ShapeDtypeStruct((K, N), jnp.float32),
            jax.ShapeDtypeStruct((K, N), jnp.int32),
            jax.ShapeDtypeStruct((N, DP), jnp.float32),
        ],
    )(scal, pos, posT, batr, batt, z2, emb_p)

    bd2 = bd2t.T
    bid = bidt.T
    d2w = bd2.reshape(N * K // 128, 128)
    distw, gw = pl.pallas_call(
        _geom_kernel,
        in_specs=[pl.BlockSpec(d2w.shape, lambda: (0, 0))],
        out_specs=[pl.BlockSpec(d2w.shape, lambda: (0, 0)),
                   pl.BlockSpec(d2w.shape, lambda: (0, 0))],
        out_shape=[jax.ShapeDtypeStruct(d2w.shape, jnp.float32),
                   jax.ShapeDtypeStruct(d2w.shape, jnp.float32)],
    )(d2w)

    diste = distw.reshape(N * K, 1)
    ge = gw.reshape(N * K, 1)
    wfc = pl.pallas_call(
        _edge_kernel,
        grid=((N * K) // EB,),
        in_specs=[
            pl.BlockSpec((EB, 1), lambda i: (i, 0)),
            pl.BlockSpec((EB, 1), lambda i: (i, 0)),
            pl.BlockSpec((RBF, 32), lambda i: (0, 0)),
            pl.BlockSpec((1, 32), lambda i: (0, 0)),
            pl.BlockSpec((32, DP), lambda i: (0, 0)),
        ],
        out_specs=pl.BlockSpec((EB, DP), lambda i: (i, 0)),
        out_shape=jax.ShapeDtypeStruct((N * K, DP), jnp.float32),
    )(diste, ge, W_rbf1, b1, w2rbf_p)

    idx3 = bid.reshape(NW, CH, 128)

    def interact(h, final):
        nbr = _sc_gather(h, idx3)
        if not final:
            return pl.pallas_call(
                _interact_kernel,
                grid=(NBLK,),
                in_specs=[
                    pl.BlockSpec((RB * K, DP), lambda i: (i, 0)),
                    pl.BlockSpec((RB * K, DP), lambda i: (i, 0)),
                    pl.BlockSpec((RB, DP), lambda i: (i, 0)),
                    pl.BlockSpec((DP, DP), lambda i: (0, 0)),
                    pl.BlockSpec((DP, DP), lambda i: (0, 0)),
                ],
                out_specs=pl.BlockSpec((RB, DP), lambda i: (i, 0)),
                out_shape=jax.ShapeDtypeStruct((N, DP), jnp.float32),
            )(nbr, wfc, h, wself_p, wmsg_p)
        return pl.pallas_call(
            _final_kernel,
            grid=(NBLK,),
            in_specs=[
                pl.BlockSpec((RB * K, DP), lambda i: (i, 0)),
                pl.BlockSpec((RB * K, DP), lambda i: (i, 0)),
                pl.BlockSpec((RB, DP), lambda i: (i, 0)),
                pl.BlockSpec((DP, DP), lambda i: (0, 0)),
                pl.BlockSpec((DP, DP), lambda i: (0, 0)),
                pl.BlockSpec((DP, 32), lambda i: (0, 0)),
                pl.BlockSpec((32, 1), lambda i: (0, 0)),
                pl.BlockSpec((RB, 1), lambda i: (i, 0)),
                pl.BlockSpec((128, 1), lambda i: (0, 0)),
                pl.BlockSpec((RB, 1), lambda i: (i, 0)),
            ],
            out_specs=pl.BlockSpec((1, G), lambda i: (0, 0)),
            out_shape=jax.ShapeDtypeStruct((1, G), jnp.float32),
        )(nbr, wfc, h, wself_p, wmsg_p, w1_p, W2, z2, shift_p, batr)

    h1 = interact(nodes, final=False)
    eng = interact(h1, final=True)
    return eng.reshape(G, 1)


# speedup vs baseline: 1.5172x; 1.2082x over previous
import functools

import jax
import jax.numpy as jnp
from jax import lax
from jax.experimental import pallas as pl
from jax.experimental.pallas import tpu as pltpu
from jax.experimental.pallas import tpu_sc as plsc

N = 4096
G = 128
K = 16
D = 28
DP = 32
RBF = 8
RC = 5.0
ZMAX = 100
RB = 128
NBLK = N // RB
RBR = 256
CW = 256
NBLKR = N // RBR
EB = 2048
BIG = 1e10
NW = 32
CH = (N * K) // NW // 128


def _topk_kernel(scal_ref, posr_ref, post_ref, batr_ref, batt_ref, z_ref,
                 emb_ref, bd2_ref, bid_ref, nodes_ref, cache_ref):
    i = pl.program_id(0)
    ws_blk = scal_ref[0, i]
    nch = scal_ref[1, i]

    rowid = i * RBR + lax.broadcasted_iota(jnp.int32, (1, RBR), 1)
    br = batt_ref[:, pl.ds(i * RBR, RBR)]
    prw = post_ref[:, pl.ds(i * RBR, RBR)]

    def cand_d2(c):
        off = (ws_blk + c) * CW
        pc = posr_ref[pl.ds(off, CW), :]
        d2 = ((pc[:, 0:1] - prw[0:1, :]) ** 2
              + (pc[:, 1:2] - prw[1:2, :]) ** 2
              + (pc[:, 2:3] - prw[2:3, :]) ** 2)
        bc = batr_ref[pl.ds(off, CW), :]
        gcol = off + lax.broadcasted_iota(jnp.int32, (CW, 1), 0)
        mask = (bc == br) & (gcol != rowid)
        return jnp.where(mask, d2, BIG), off

    bd2_ref[...] = jnp.full((K, RBR), BIG, jnp.float32)
    IBIG = jnp.int32(2**30)

    def chunk_body(c, carry):
        cand, _ = cand_d2(c)
        cache_ref[pl.ds(c * CW, CW), :] = cand
        comb = jnp.concatenate([bd2_ref[...], cand], axis=0)
        ms = []
        for k in range(K):
            m = jnp.min(comb, axis=0, keepdims=True)
            ms.append(m)
            comb = jnp.where(comb == m, BIG, comb)
        bd2_ref[...] = jnp.concatenate(ms, axis=0)
        return carry

    lax.fori_loop(0, nch, chunk_body, 0)

    nid_ref = bid_ref
    nid_ref[...] = jnp.full((K, RBR), IBIG, jnp.int32)
    bd = bd2_ref[...]

    def id_body(c, carry):
        cand = cache_ref[pl.ds(c * CW, CW), :]
        off = (ws_blk + c) * CW
        gcs = off + lax.broadcasted_iota(jnp.int32, (CW, 1), 0)
        ids = []
        for k in range(K):
            mk = cand == bd[k:k + 1, :]
            ids.append(jnp.min(jnp.where(mk, gcs, IBIG), axis=0,
                               keepdims=True))
        idcat = jnp.concatenate(ids, axis=0)
        nid_ref[...] = jnp.minimum(nid_ref[...], idcat)
        return carry

    lax.fori_loop(0, nch, id_body, 0)
    bid_ref[...] = jnp.clip(nid_ref[...], 0, N - 1)

    zoh = (z_ref[...] == lax.broadcasted_iota(jnp.int32, (1, 128), 1)
           ).astype(jnp.float32)
    nodes_ref[...] = jnp.dot(zoh, emb_ref[...],
                             precision=lax.Precision.HIGHEST,
                             preferred_element_type=jnp.float32)


def _geom_kernel(d2_ref, dist_ref, g_ref):
    d2 = d2_ref[...]
    dist = jnp.sqrt(d2 + 1e-12)
    dc = jnp.minimum(dist, RC)
    fc = 0.5 * (jnp.cos(jnp.pi * dc / RC) + 1.0)
    valid = (d2 < 1e9).astype(jnp.float32)
    dist_ref[...] = dist
    g_ref[...] = fc * valid


def _edge_kernel(dist_ref, g_ref, w1_ref, b1_ref, w2_ref, out_ref):
    dist = dist_ref[...]
    cen = (RC / (RBF - 1)) * lax.broadcasted_iota(jnp.int32, (1, RBF), 1
                                                  ).astype(jnp.float32)
    rbf = jnp.exp(-10.0 * (dist - cen) ** 2)
    h = jax.nn.silu(jnp.dot(rbf, w1_ref[...],
                            preferred_element_type=jnp.float32) + b1_ref[...])
    we = jnp.dot(h, w2_ref[...], preferred_element_type=jnp.float32)
    out_ref[...] = we * g_ref[...]


def _interact_kernel(nbr_ref, wfc_ref, h_ref, ws_ref, wm_ref, out_ref):
    prod = nbr_ref[...] * wfc_ref[...]
    msg = jnp.sum(prod.reshape(RB, K, DP), axis=1)
    h = h_ref[...]
    out_ref[...] = jax.nn.silu(
        jnp.dot(h, ws_ref[...], preferred_element_type=jnp.float32)
        + jnp.dot(msg, wm_ref[...], preferred_element_type=jnp.float32))


def _final_kernel(nbr_ref, wfc_ref, h_ref, ws_ref, wm_ref, w1_ref, w2_ref,
                  z_ref, shift_ref, batr_ref, out_ref):
    i = pl.program_id(0)
    prod = nbr_ref[...] * wfc_ref[...]
    msg = jnp.sum(prod.reshape(RB, K, DP), axis=1)
    h = h_ref[...]
    h2 = jax.nn.silu(
        jnp.dot(h, ws_ref[...], preferred_element_type=jnp.float32)
        + jnp.dot(msg, wm_ref[...], preferred_element_type=jnp.float32))
    t = jax.nn.silu(jnp.dot(h2, w1_ref[...],
                            preferred_element_type=jnp.float32))
    ae = jnp.dot(t, w2_ref[...], preferred_element_type=jnp.float32)
    zoh = (z_ref[...] == lax.broadcasted_iota(jnp.int32, (1, 128), 1)
           ).astype(jnp.float32)
    ae = ae + jnp.dot(zoh, shift_ref[...], precision=lax.Precision.HIGHEST,
                      preferred_element_type=jnp.float32)
    boh = (batr_ref[...] == lax.broadcasted_iota(jnp.int32, (1, G), 1)
           ).astype(jnp.float32)
    contrib = jnp.sum(boh * ae, axis=0, keepdims=True)

    @pl.when(i == 0)
    def _():
        out_ref[...] = jnp.zeros((1, G), jnp.float32)

    out_ref[...] += contrib


def _sc_gather(table, idx3):
    mesh = plsc.VectorSubcoreMesh(core_axis_name="c", subcore_axis_name="s")

    @functools.partial(
        pl.kernel,
        out_type=jax.ShapeDtypeStruct((NW * CH * 128, DP), jnp.float32),
        mesh=mesh,
        compiler_params=pltpu.CompilerParams(use_tc_tiling_on_sc=False),
        scratch_types=[
            pltpu.VMEM((CH, 128), jnp.int32),
            pltpu.VMEM((CH * 128, DP), jnp.float32),
            pltpu.SemaphoreType.DMA,
        ],
    )
    def gk(table_hbm, idx_hbm, out_hbm, idx_v, rows_v, sem):
        w = lax.axis_index("s") * 2 + lax.axis_index("c")
        pltpu.sync_copy(idx_hbm.at[w], idx_v)
        copies = [
            pltpu.async_copy(table_hbm.at[idx_v.at[j]],
                             rows_v.at[pl.ds(j * 128, 128)], sem)
            for j in range(CH)
        ]
        for c in copies:
            c.wait()
        pltpu.sync_copy(rows_v, out_hbm.at[pl.ds(w * CH * 128, CH * 128)])

    return gk(table, idx3)


def kernel(z, pos, batch, emb, W_rbf1, b_rbf1, W_rbf2, W_self, W_msg, W1, W2,
           shift):
    pos = pos.astype(jnp.float32)
    posT = pos.T
    batch = batch.astype(jnp.int32)
    batr = batch.reshape(N, 1)
    batt = batch.reshape(1, N)
    z2 = z.astype(jnp.int32).reshape(N, 1)

    emb_p = jnp.zeros((128, DP), jnp.float32).at[:ZMAX, :D].set(emb)
    w2rbf_p = jnp.zeros((32, DP), jnp.float32).at[:, :D].set(W_rbf2)
    wself_p = jnp.zeros((DP, DP), jnp.float32).at[:D, :D].set(W_self)
    wmsg_p = jnp.zeros((DP, DP), jnp.float32).at[:D, :D].set(W_msg)
    w1_p = jnp.zeros((DP, 32), jnp.float32).at[:D, :].set(W1)
    shift_p = jnp.zeros((128, 1), jnp.float32).at[:ZMAX, :].set(shift)
    b1 = b_rbf1.reshape(1, 32)

    gids = jnp.arange(G, dtype=jnp.int32)
    starts = jnp.searchsorted(batch, gids, side="left").astype(jnp.int32)
    ends = jnp.searchsorted(batch, gids, side="right").astype(jnp.int32)
    bfirst = batch.reshape(NBLKR, RBR)[:, 0]
    blast = batch.reshape(NBLKR, RBR)[:, -1]
    ws_blk = starts[bfirst] // CW
    nch = (ends[blast] - 1) // CW - ws_blk + 1
    scal = jnp.stack([ws_blk, nch]).astype(jnp.int32)

    full = lambda shape: pl.BlockSpec(shape, lambda i, s: (0, 0))
    rowblk = lambda shape: pl.BlockSpec(shape, lambda i, s: (i, 0))

    colblk = lambda shape: pl.BlockSpec(shape, lambda i, s: (0, i))
    bd2t, bidt, nodes = pl.pallas_call(
        _topk_kernel,
        grid_spec=pltpu.PrefetchScalarGridSpec(
            num_scalar_prefetch=1,
            grid=(NBLKR,),
            in_specs=[
                full((N, 3)),
                full((3, N)),
                full((N, 1)),
                full((1, N)),
                rowblk((RBR, 1)),
                full((128, DP)),
            ],
            out_specs=[
                colblk((K, RBR)),
                colblk((K, RBR)),
                rowblk((RBR, DP)),
            ],
            scratch_shapes=[pltpu.VMEM((N, RBR), jnp.float32)],
        ),
        out_shape=[
            jax.ShapeDtypeStruct((K, N), jnp.float32),
            jax.ShapeDtypeStruct((K, N), jnp.int32),
            jax.ShapeDtypeStruct((N, DP), jnp.float32),
        ],
    )(scal, pos, posT, batr, batt, z2, emb_p)

    bd2 = bd2t.T
    bid = bidt.T
    d2w = bd2.reshape(N * K // 128, 128)
    distw, gw = pl.pallas_call(
        _geom_kernel,
        in_specs=[pl.BlockSpec(d2w.shape, lambda: (0, 0))],
        out_specs=[pl.BlockSpec(d2w.shape, lambda: (0, 0)),
                   pl.BlockSpec(d2w.shape, lambda: (0, 0))],
        out_shape=[jax.ShapeDtypeStruct(d2w.shape, jnp.float32),
                   jax.ShapeDtypeStruct(d2w.shape, jnp.float32)],
    )(d2w)

    diste = distw.reshape(N * K, 1)
    ge = gw.reshape(N * K, 1)
    wfc = pl.pallas_call(
        _edge_kernel,
        grid=((N * K) // EB,),
        in_specs=[
            pl.BlockSpec((EB, 1), lambda i: (i, 0)),
            pl.BlockSpec((EB, 1), lambda i: (i, 0)),
            pl.BlockSpec((RBF, 32), lambda i: (0, 0)),
            pl.BlockSpec((1, 32), lambda i: (0, 0)),
            pl.BlockSpec((32, DP), lambda i: (0, 0)),
        ],
        out_specs=pl.BlockSpec((EB, DP), lambda i: (i, 0)),
        out_shape=jax.ShapeDtypeStruct((N * K, DP), jnp.float32),
    )(diste, ge, W_rbf1, b1, w2rbf_p)

    idx3 = bid.reshape(NW, CH, 128)

    def interact(h, final):
        nbr = _sc_gather(h, idx3)
        if not final:
            return pl.pallas_call(
                _interact_kernel,
                grid=(NBLK,),
                in_specs=[
                    pl.BlockSpec((RB * K, DP), lambda i: (i, 0)),
                    pl.BlockSpec((RB * K, DP), lambda i: (i, 0)),
                    pl.BlockSpec((RB, DP), lambda i: (i, 0)),
                    pl.BlockSpec((DP, DP), lambda i: (0, 0)),
                    pl.BlockSpec((DP, DP), lambda i: (0, 0)),
                ],
                out_specs=pl.BlockSpec((RB, DP), lambda i: (i, 0)),
                out_shape=jax.ShapeDtypeStruct((N, DP), jnp.float32),
            )(nbr, wfc, h, wself_p, wmsg_p)
        return pl.pallas_call(
            _final_kernel,
            grid=(NBLK,),
            in_specs=[
                pl.BlockSpec((RB * K, DP), lambda i: (i, 0)),
                pl.BlockSpec((RB * K, DP), lambda i: (i, 0)),
                pl.BlockSpec((RB, DP), lambda i: (i, 0)),
                pl.BlockSpec((DP, DP), lambda i: (0, 0)),
                pl.BlockSpec((DP, DP), lambda i: (0, 0)),
                pl.BlockSpec((DP, 32), lambda i: (0, 0)),
                pl.BlockSpec((32, 1), lambda i: (0, 0)),
                pl.BlockSpec((RB, 1), lambda i: (i, 0)),
                pl.BlockSpec((128, 1), lambda i: (0, 0)),
                pl.BlockSpec((RB, 1), lambda i: (i, 0)),
            ],
            out_specs=pl.BlockSpec((1, G), lambda i: (0, 0)),
            out_shape=jax.ShapeDtypeStruct((1, G), jnp.float32),
        )(nbr, wfc, h, wself_p, wmsg_p, w1_p, W2, z2, shift_p, batr)

    h1 = interact(nodes, final=False)
    eng = interact(h1, final=True)
    return eng.reshape(G, 1)
